# Initial kernel scaffold; baseline (speedup 1.0000x reference)
import jax
import jax.numpy as jnp
from jax.experimental import pallas as pl


def kernel(z, edge_index, weight1, gru1_wih, gru1_whh, gru1_bih, gru1_bhh, weight2, gru2_wih, gru2_whh, gru2_bih, gru2_bhh, lin_w, lin_b):
    raise NotImplementedError("write your pallas kernel here")


# speedup vs baseline: 6.3083x; 6.3083x over previous
import functools

import jax
import jax.numpy as jnp
from jax import lax
from jax.experimental import pallas as pl
from jax.experimental.pallas import tpu as pltpu
from jax.experimental.pallas import tpu_sc as plsc

N = 10000
E = 320000
D = 128
L = 8

NC = 2
NS = 16
NW = NC * NS
EPW = E // NW
CH = 80
NCHUNK = EPW // CH
NP = 10240
RPT = NP // NS


def _sc_scatter_body(m_hbm, src_hbm, dst_hbm, zero_hbm, out_hbm,
                     src_v, dst_v, rows0, agg_sh, sem0):
    cid = lax.axis_index("c")
    sid = lax.axis_index("s")
    wid = sid * NC + cid

    pltpu.sync_copy(zero_hbm, agg_sh.at[pl.ds(sid * RPT, RPT)])
    pltpu.sync_copy(src_hbm.at[wid], src_v)
    pltpu.sync_copy(dst_hbm.at[wid], dst_v)
    plsc.subcore_barrier()

    def body(j, _):
        pltpu.async_copy(m_hbm.at[src_v.at[j]], rows0, sem0).wait()
        pltpu.sync_copy(rows0, agg_sh.at[dst_v.at[j]], add=True)
        return 0

    lax.fori_loop(0, NCHUNK, body, 0)

    plsc.subcore_barrier()
    pltpu.sync_copy(agg_sh.at[pl.ds(sid * RPT, RPT)],
                    out_hbm.at[cid, pl.ds(sid * RPT, RPT)])


@functools.cache
def _make_sc_scatter():
    mesh = plsc.VectorSubcoreMesh(
        core_axis_name="c", subcore_axis_name="s",
        num_cores=NC, num_subcores=NS)
    return pl.kernel(
        _sc_scatter_body,
        out_type=jax.ShapeDtypeStruct((NC, NP, D), jnp.float32),
        mesh=mesh,
        scratch_types=[
            pltpu.VMEM((NCHUNK, CH), jnp.int32),
            pltpu.VMEM((NCHUNK, CH), jnp.int32),
            pltpu.VMEM((CH, D), jnp.float32),
            pltpu.VMEM_SHARED((NP, D), jnp.float32),
            pltpu.SemaphoreType.DMA,
        ],
    )


def _sc_scatter(m, src_r, dst_r, zero_rows):
    return _make_sc_scatter()(m, src_r, dst_r, zero_rows)


BN = 1024


def _mm_body(x_ref, w_ref, o_ref):
    o_ref[...] = jnp.dot(x_ref[...], w_ref[...],
                         preferred_element_type=jnp.float32)


def _mm(x, w):
    return pl.pallas_call(
        _mm_body,
        grid=(NP // BN,),
        in_specs=[
            pl.BlockSpec((BN, D), lambda i: (i, 0)),
            pl.BlockSpec((D, D), lambda i: (0, 0)),
        ],
        out_specs=pl.BlockSpec((BN, D), lambda i: (i, 0)),
        out_shape=jax.ShapeDtypeStruct((NP, D), jnp.float32),
    )(x, w)


def _gru_body(apply_tanh, aggp_ref, h_ref, wih_ref, whh_ref, bih_ref,
              bhh_ref, wn_ref, h_out, m_out):
    agg = aggp_ref[0] + aggp_ref[1]
    h = h_ref[...]
    gi = jnp.dot(agg, wih_ref[...], preferred_element_type=jnp.float32)
    gi = gi + bih_ref[...]
    gh = jnp.dot(h, whh_ref[...], preferred_element_type=jnp.float32)
    gh = gh + bhh_ref[...]
    r = jax.nn.sigmoid(gi[:, :D] + gh[:, :D])
    zg = jax.nn.sigmoid(gi[:, D:2 * D] + gh[:, D:2 * D])
    n = jnp.tanh(gi[:, 2 * D:] + r * gh[:, 2 * D:])
    hn = (1.0 - zg) * n + zg * h
    if apply_tanh:
        hn = jnp.tanh(hn)
    h_out[...] = hn
    m_out[...] = jnp.dot(hn, wn_ref[...], preferred_element_type=jnp.float32)


def _gru_step(aggp, h, wihT, whhT, bih, bhh, wnext, apply_tanh):
    return pl.pallas_call(
        functools.partial(_gru_body, apply_tanh),
        grid=(NP // BN,),
        in_specs=[
            pl.BlockSpec((NC, BN, D), lambda i: (0, i, 0)),
            pl.BlockSpec((BN, D), lambda i: (i, 0)),
            pl.BlockSpec((D, 3 * D), lambda i: (0, 0)),
            pl.BlockSpec((D, 3 * D), lambda i: (0, 0)),
            pl.BlockSpec((1, 3 * D), lambda i: (0, 0)),
            pl.BlockSpec((1, 3 * D), lambda i: (0, 0)),
            pl.BlockSpec((D, D), lambda i: (0, 0)),
        ],
        out_specs=[
            pl.BlockSpec((BN, D), lambda i: (i, 0)),
            pl.BlockSpec((BN, D), lambda i: (i, 0)),
        ],
        out_shape=[
            jax.ShapeDtypeStruct((NP, D), jnp.float32),
            jax.ShapeDtypeStruct((NP, D), jnp.float32),
        ],
    )(aggp, h, wihT, whhT, bih, bhh, wnext)


def kernel(z, edge_index, weight1, gru1_wih, gru1_whh, gru1_bih, gru1_bhh,
           weight2, gru2_wih, gru2_whh, gru2_bih, gru2_bhh, lin_w, lin_b):
    ei = edge_index.astype(jnp.int32)
    src_r = ei[0].reshape(NW, NCHUNK, CH)
    dst_r = ei[1].reshape(NW, NCHUNK, CH)
    zero_rows = jnp.zeros((RPT, D), jnp.float32)

    w1ihT = gru1_wih.T
    w1hhT = gru1_whh.T
    b1ih = gru1_bih.reshape(1, 3 * D)
    b1hh = gru1_bhh.reshape(1, 3 * D)
    w2ihT = gru2_wih.T
    w2hhT = gru2_whh.T
    b2ih = gru2_bih.reshape(1, 3 * D)
    b2hh = gru2_bhh.reshape(1, 3 * D)
    lin_w_pad = jnp.pad(lin_w, ((0, 0), (0, D - 1)))

    h = jnp.pad(z, ((0, NP - N), (0, 0)))
    m = _mm(h, weight1[0])
    for i in range(L):
        aggp = _sc_scatter(m, src_r, dst_r, zero_rows)
        last = i == L - 1
        wnext = weight2[0] if last else weight1[i + 1]
        h, m = _gru_step(aggp, h, w1ihT, w1hhT, b1ih, b1hh, wnext, last)
    for i in range(L):
        aggp = _sc_scatter(m, src_r, dst_r, zero_rows)
        last = i == L - 1
        wnext = lin_w_pad if last else weight2[i + 1]
        h, m = _gru_step(aggp, h, w2ihT, w2hhT, b2ih, b2hh, wnext, last)
    return m[:N, :1] + lin_b
